# baseline (speedup 1.0000x reference)
import functools
import jax
import jax.numpy as jnp
from jax import lax
from jax.experimental import pallas as pl
from jax.experimental.pallas import tpu as pltpu
from jax.experimental.pallas import tpu_sc as plsc

N = 8192
DIM = 128
NC = 8
B = 8

SC_CORES = 2
SC_SUBCORES = 16
LANES = 16
NW = SC_CORES * SC_SUBCORES
COLS = N // NW
GRPS = COLS // LANES
CHUNK = 128
NCHUNK = N // CHUNK
SLOTS = B + 1
ACCW = SLOTS * COLS


def _sc_segsum(ed_hbm, adj_hbm, out_hbm, edv, idxv, bufv, accv, sem0, sem1):
    cid = lax.axis_index("c")
    sid = lax.axis_index("s")
    w = sid * SC_CORES + cid

    pltpu.sync_copy(ed_hbm, edv)
    eds = [edv[pl.ds(k * LANES, LANES)][0] for k in range(B)]

    def seg_scalar(r):
        s = jnp.int32(0)
        for k in range(B):
            s = s + jnp.where(eds[k] <= r, jnp.int32(1), jnp.int32(0))
        return s

    def _zero(g, carry):
        accv[pl.ds(g * LANES, LANES)] = jnp.zeros((LANES,), jnp.float32)
        return carry

    lax.fori_loop(0, ACCW // LANES, _zero, 0)

    iota = lax.iota(jnp.int32, LANES)

    def fill_idx(bslot, c):
        for g in range(CHUNK // LANES):
            rid = c * CHUNK + g * LANES + iota
            idxv[bslot, pl.ds(g * LANES, LANES)] = rid * NW + w

    def start(bslot, sem):
        return pltpu.async_copy(adj_hbm.at[idxv.at[bslot]], bufv.at[bslot], sem)

    def wait(bslot, sem):
        pltpu.make_async_copy(adj_hbm.at[idxv.at[bslot]], bufv.at[bslot], sem).wait()

    sems = (sem0, sem1)

    fill_idx(0, 0)
    start(0, sems[0])

    def process(c, bslot):
        @pl.when(c + 1 < NCHUNK)
        def _pref():
            fill_idx(1 - bslot, c + 1)
            start(1 - bslot, sems[1 - bslot])

        wait(bslot, sems[bslot])

        s0 = seg_scalar(c * CHUNK)
        s1 = seg_scalar(c * CHUNK + (CHUNK - 1))

        @pl.when(s0 == s1)
        def _fast():
            def rowf(j, acc):
                return tuple(
                    acc[g] + bufv[bslot, j, pl.ds(g * LANES, LANES)]
                    for g in range(GRPS)
                )

            acc = lax.fori_loop(
                0, CHUNK, rowf,
                tuple(jnp.zeros((LANES,), jnp.float32) for _ in range(GRPS)),
            )
            off = s0 * COLS
            for g in range(GRPS):
                accv[pl.ds(off + g * LANES, LANES)] = (
                    accv[pl.ds(off + g * LANES, LANES)] + acc[g]
                )

        @pl.when(s0 != s1)
        def _slow():
            def rows_(j, carry):
                off = seg_scalar(c * CHUNK + j) * COLS
                for g in range(GRPS):
                    accv[pl.ds(off + g * LANES, LANES)] = (
                        accv[pl.ds(off + g * LANES, LANES)]
                        + bufv[bslot, j, pl.ds(g * LANES, LANES)]
                    )
                return carry

            lax.fori_loop(0, CHUNK, rows_, 0)

    def outer(cc, carry):
        process(cc * 2, 0)
        process(cc * 2 + 1, 1)
        return carry

    lax.fori_loop(0, NCHUNK // 2, outer, 0)

    pltpu.sync_copy(accv, out_hbm.at[w])


def _tc_finish(x_ref, W_ref, b_ref, parts_ref, out_ref):
    hidden = (
        jnp.dot(x_ref[...], W_ref[...], preferred_element_type=jnp.float32)
        + b_ref[...]
    )
    acc = jnp.zeros((B, NC), jnp.float32)
    for w in range(NW):
        acc = acc + jnp.dot(
            parts_ref[w, 0:B, :],
            hidden[w * COLS:(w + 1) * COLS, :],
            preferred_element_type=jnp.float32,
        )
    out_ref[...] = acc


def kernel(x, ed_idx, adj, W, b):
    ed_b = jnp.broadcast_to(ed_idx[:, None], (B, LANES)).reshape(B * LANES)
    adj_r = adj.reshape(N * NW, COLS)

    sc = functools.partial(
        pl.kernel,
        out_type=jax.ShapeDtypeStruct((NW, ACCW), jnp.float32),
        mesh=plsc.VectorSubcoreMesh(
            core_axis_name="c", subcore_axis_name="s",
            num_cores=SC_CORES, num_subcores=SC_SUBCORES,
        ),
        scratch_types=[
            pltpu.VMEM((B * LANES,), jnp.int32),
            pltpu.VMEM((2, CHUNK), jnp.int32),
            pltpu.VMEM((2, CHUNK, COLS), jnp.float32),
            pltpu.VMEM((ACCW,), jnp.float32),
            pltpu.SemaphoreType.DMA,
            pltpu.SemaphoreType.DMA,
        ],
    )(_sc_segsum)
    parts = sc(ed_b, adj_r).reshape(NW, SLOTS, COLS)

    return pl.pallas_call(
        _tc_finish,
        out_shape=jax.ShapeDtypeStruct((B, NC), jnp.float32),
    )(x, W, b.reshape(1, NC), parts)


# speedup vs baseline: 2.4814x; 2.4814x over previous
import functools
import jax
import jax.numpy as jnp
from jax import lax
from jax.experimental import pallas as pl
from jax.experimental.pallas import tpu as pltpu
from jax.experimental.pallas import tpu_sc as plsc

N = 8192
DIM = 128
NC = 8
B = 8

SC_CORES = 2
SC_SUBCORES = 16
LANES = 16
NW = SC_CORES * SC_SUBCORES
COLS = N // NW
GRPS = COLS // LANES
CHUNK = 128
NCHUNK = N // CHUNK
SLOTS = B + 1
ACCW = SLOTS * COLS


def _sc_segsum(ed_hbm, adj_hbm, out_hbm, edv, bufv, accv, sem0, sem1):
    cid = lax.axis_index("c")
    sid = lax.axis_index("s")
    w = sid * SC_CORES + cid

    pltpu.sync_copy(ed_hbm, edv)
    eds = [edv[pl.ds(k * LANES, LANES)][0] for k in range(B)]

    def seg_scalar(r):
        s = jnp.int32(0)
        for k in range(B):
            s = s + jnp.where(eds[k] <= r, jnp.int32(1), jnp.int32(0))
        return s

    def _zero(g, carry):
        accv[pl.ds(g * LANES, LANES)] = jnp.zeros((LANES,), jnp.float32)
        return carry

    lax.fori_loop(0, ACCW // LANES, _zero, 0)

    iota = lax.iota(jnp.int32, LANES)

    col0 = w * COLS

    def src_slice(c):
        return adj_hbm.at[pl.ds(c * CHUNK, CHUNK), pl.ds(col0, COLS)]

    def start(bslot, c, sem):
        return pltpu.async_copy(src_slice(c), bufv.at[bslot], sem)

    def wait(bslot, c, sem):
        pltpu.make_async_copy(src_slice(c), bufv.at[bslot], sem).wait()

    sems = (sem0, sem1)

    start(0, 0, sems[0])

    def process(c, bslot):
        @pl.when(c + 1 < NCHUNK)
        def _pref():
            start(1 - bslot, c + 1, sems[1 - bslot])

        wait(bslot, c, sems[bslot])

        s0 = seg_scalar(c * CHUNK)
        s1 = seg_scalar(c * CHUNK + (CHUNK - 1))

        @pl.when(s0 == s1)
        def _fast():
            def rowf(jj, acc):
                j = jj * 4
                for u in range(4):
                    acc = tuple(
                        acc[g] + bufv[bslot, j + u, pl.ds(g * LANES, LANES)]
                        for g in range(GRPS)
                    )
                return acc

            acc = lax.fori_loop(
                0, CHUNK // 4, rowf,
                tuple(jnp.zeros((LANES,), jnp.float32) for _ in range(GRPS)),
            )
            off = s0 * COLS
            for g in range(GRPS):
                accv[pl.ds(off + g * LANES, LANES)] = (
                    accv[pl.ds(off + g * LANES, LANES)] + acc[g]
                )

        @pl.when(s0 != s1)
        def _slow():
            def rows_(j, carry):
                off = seg_scalar(c * CHUNK + j) * COLS
                for g in range(GRPS):
                    accv[pl.ds(off + g * LANES, LANES)] = (
                        accv[pl.ds(off + g * LANES, LANES)]
                        + bufv[bslot, j, pl.ds(g * LANES, LANES)]
                    )
                return carry

            lax.fori_loop(0, CHUNK, rows_, 0)

    def outer(cc, carry):
        process(cc * 2, 0)
        process(cc * 2 + 1, 1)
        return carry

    lax.fori_loop(0, NCHUNK // 2, outer, 0)

    pltpu.sync_copy(accv, out_hbm.at[w])


def _tc_finish(x_ref, W_ref, b_ref, parts_ref, out_ref):
    hidden = (
        jnp.dot(x_ref[...], W_ref[...], preferred_element_type=jnp.float32)
        + b_ref[...]
    )
    acc = jnp.zeros((B, NC), jnp.float32)
    for w in range(NW):
        acc = acc + jnp.dot(
            parts_ref[w, 0:B, :],
            hidden[w * COLS:(w + 1) * COLS, :],
            preferred_element_type=jnp.float32,
        )
    out_ref[...] = acc


def kernel(x, ed_idx, adj, W, b):
    ed_b = jnp.broadcast_to(ed_idx[:, None], (B, LANES)).reshape(B * LANES)

    sc = functools.partial(
        pl.kernel,
        out_type=jax.ShapeDtypeStruct((NW, ACCW), jnp.float32),
        mesh=plsc.VectorSubcoreMesh(
            core_axis_name="c", subcore_axis_name="s",
            num_cores=SC_CORES, num_subcores=SC_SUBCORES,
        ),
        scratch_types=[
            pltpu.VMEM((B * LANES,), jnp.int32),
            pltpu.VMEM((2, CHUNK, COLS), jnp.float32),
            pltpu.VMEM((ACCW,), jnp.float32),
            pltpu.SemaphoreType.DMA,
            pltpu.SemaphoreType.DMA,
        ],
    )(_sc_segsum)
    parts = sc(ed_b, adj).reshape(NW, SLOTS, COLS)

    return pl.pallas_call(
        _tc_finish,
        out_shape=jax.ShapeDtypeStruct((B, NC), jnp.float32),
    )(x, W, b.reshape(1, NC), parts)


# speedup vs baseline: 4.1342x; 1.6661x over previous
import functools
import jax
import jax.numpy as jnp
from jax import lax
from jax.experimental import pallas as pl
from jax.experimental.pallas import tpu as pltpu
from jax.experimental.pallas import tpu_sc as plsc

N = 8192
DIM = 128
NC = 8
B = 8

ROWS = 128
SPLIT = 5120
STEPS_TC = SPLIT // ROWS

SC_CORES = 2
SC_SUBCORES = 16
LANES = 16
NW = SC_CORES * SC_SUBCORES
COLS = N // NW
GRPS = COLS // LANES
CHUNK = 128
NCHUNK = (N - SPLIT) // CHUNK
SLOTS = B + 1
ACCW = SLOTS * COLS


def _sc_segsum(ed_hbm, adj_hbm, out_hbm, edv, bufv, accv, sem0, sem1):
    cid = lax.axis_index("c")
    sid = lax.axis_index("s")
    w = sid * SC_CORES + cid

    pltpu.sync_copy(ed_hbm, edv)
    eds = [edv[pl.ds(k * LANES, LANES)][0] for k in range(B)]

    def seg_scalar(r):
        s = jnp.int32(0)
        for k in range(B):
            s = s + jnp.where(eds[k] <= r, jnp.int32(1), jnp.int32(0))
        return s

    def _zero(g, carry):
        accv[pl.ds(g * LANES, LANES)] = jnp.zeros((LANES,), jnp.float32)
        return carry

    lax.fori_loop(0, ACCW // LANES, _zero, 0)

    col0 = w * COLS

    def src_slice(c):
        return adj_hbm.at[pl.ds(SPLIT + c * CHUNK, CHUNK), pl.ds(col0, COLS)]

    def start(bslot, c, sem):
        return pltpu.async_copy(src_slice(c), bufv.at[bslot], sem)

    def wait(bslot, c, sem):
        pltpu.make_async_copy(src_slice(c), bufv.at[bslot], sem).wait()

    sems = (sem0, sem1)

    start(0, 0, sems[0])

    def process(c, bslot):
        @pl.when(c + 1 < NCHUNK)
        def _pref():
            start(1 - bslot, c + 1, sems[1 - bslot])

        wait(bslot, c, sems[bslot])

        r0 = SPLIT + c * CHUNK
        s0 = seg_scalar(r0)
        s1 = seg_scalar(r0 + (CHUNK - 1))

        @pl.when(s0 == s1)
        def _fast():
            def rowf(jj, acc):
                j = jj * 4
                for u in range(4):
                    acc = tuple(
                        acc[g] + bufv[bslot, j + u, pl.ds(g * LANES, LANES)]
                        for g in range(GRPS)
                    )
                return acc

            acc = lax.fori_loop(
                0, CHUNK // 4, rowf,
                tuple(jnp.zeros((LANES,), jnp.float32) for _ in range(GRPS)),
            )
            off = s0 * COLS
            for g in range(GRPS):
                accv[pl.ds(off + g * LANES, LANES)] = (
                    accv[pl.ds(off + g * LANES, LANES)] + acc[g]
                )

        @pl.when(s0 != s1)
        def _slow():
            def rows_(j, carry):
                off = seg_scalar(r0 + j) * COLS
                for g in range(GRPS):
                    accv[pl.ds(off + g * LANES, LANES)] = (
                        accv[pl.ds(off + g * LANES, LANES)]
                        + bufv[bslot, j, pl.ds(g * LANES, LANES)]
                    )
                return carry

            lax.fori_loop(0, CHUNK, rows_, 0)

    def outer(cc, carry):
        process(cc * 2, 0)
        process(cc * 2 + 1, 1)
        return carry

    lax.fori_loop(0, NCHUNK // 2, outer, 0)

    pltpu.sync_copy(accv, out_hbm.at[w])


def _tc_partial(ed_ref, x_ref, W_ref, b_ref, adj_ref, out_ref, acc_ref, hid_ref):
    i = pl.program_id(0)

    @pl.when(i == 0)
    def _init():
        acc_ref[...] = jnp.zeros_like(acc_ref)
        hid_ref[...] = (
            jnp.dot(x_ref[...], W_ref[...], preferred_element_type=jnp.float32)
            + b_ref[...]
        )

    base = i * ROWS
    s0 = jnp.int32(0)
    s1 = jnp.int32(0)
    for k in range(B):
        e = ed_ref[k]
        s0 = s0 + jnp.where(e <= base, 1, 0).astype(jnp.int32)
        s1 = s1 + jnp.where(e <= base + ROWS - 1, 1, 0).astype(jnp.int32)

    @pl.when(s0 == s1)
    def _fast():
        col = jnp.sum(adj_ref[...], axis=0, keepdims=True)
        acc_ref[pl.ds(s0, 1), :] += col

    @pl.when(s0 != s1)
    def _slow():
        rid = base + jax.lax.broadcasted_iota(jnp.int32, (ROWS, 1), 0)
        cnt = jnp.zeros((ROWS, 1), jnp.int32)
        for k in range(B):
            cnt = cnt + jnp.where(ed_ref[k] <= rid, 1, 0).astype(jnp.int32)
        tile = adj_ref[...]
        for s in range(B + 1):
            mask = (cnt == s).astype(jnp.float32)
            acc_ref[pl.ds(s, 1), :] += jnp.sum(tile * mask, axis=0, keepdims=True)

    @pl.when(i == STEPS_TC - 1)
    def _final():
        out_ref[...] = jnp.dot(
            acc_ref[0:B, :], hid_ref[...], preferred_element_type=jnp.float32
        )


def _tc_finish(x_ref, W_ref, b_ref, parts_ref, tc_ref, out_ref):
    hidden = (
        jnp.dot(x_ref[...], W_ref[...], preferred_element_type=jnp.float32)
        + b_ref[...]
    )
    acc = tc_ref[...]
    for w in range(NW):
        acc = acc + jnp.dot(
            parts_ref[w, 0:B, :],
            hidden[w * COLS:(w + 1) * COLS, :],
            preferred_element_type=jnp.float32,
        )
    out_ref[...] = acc


def kernel(x, ed_idx, adj, W, b):
    ed_b = jnp.broadcast_to(ed_idx[:, None], (B, LANES)).reshape(B * LANES)
    ed16 = jnp.concatenate([ed_idx, jnp.full((8,), N, jnp.int32)])
    b2 = b.reshape(1, NC)

    sc = functools.partial(
        pl.kernel,
        out_type=jax.ShapeDtypeStruct((NW, ACCW), jnp.float32),
        mesh=plsc.VectorSubcoreMesh(
            core_axis_name="c", subcore_axis_name="s",
            num_cores=SC_CORES, num_subcores=SC_SUBCORES,
        ),
        scratch_types=[
            pltpu.VMEM((B * LANES,), jnp.int32),
            pltpu.VMEM((2, CHUNK, COLS), jnp.float32),
            pltpu.VMEM((ACCW,), jnp.float32),
            pltpu.SemaphoreType.DMA,
            pltpu.SemaphoreType.DMA,
        ],
    )(_sc_segsum)
    parts = sc(ed_b, adj).reshape(NW, SLOTS, COLS)

    grid_spec = pltpu.PrefetchScalarGridSpec(
        num_scalar_prefetch=1,
        grid=(STEPS_TC,),
        in_specs=[
            pl.BlockSpec((N, DIM), lambda i, *_: (0, 0)),
            pl.BlockSpec((DIM, NC), lambda i, *_: (0, 0)),
            pl.BlockSpec((1, NC), lambda i, *_: (0, 0)),
            pl.BlockSpec((ROWS, N), lambda i, *_: (i, 0)),
        ],
        out_specs=pl.BlockSpec((B, NC), lambda i, *_: (0, 0)),
        scratch_shapes=[
            pltpu.VMEM((16, N), jnp.float32),
            pltpu.VMEM((N, NC), jnp.float32),
        ],
    )
    tc_out = pl.pallas_call(
        _tc_partial,
        grid_spec=grid_spec,
        out_shape=jax.ShapeDtypeStruct((B, NC), jnp.float32),
    )(ed16, x, W, b2, adj)

    return pl.pallas_call(
        _tc_finish,
        out_shape=jax.ShapeDtypeStruct((B, NC), jnp.float32),
    )(x, W, b2, parts, tc_out)


# speedup vs baseline: 4.2421x; 1.0261x over previous
import functools
import jax
import jax.numpy as jnp
from jax import lax
from jax.experimental import pallas as pl
from jax.experimental.pallas import tpu as pltpu
from jax.experimental.pallas import tpu_sc as plsc

N = 8192
DIM = 128
NC = 8
B = 8

ROWS = 128
SPLIT = 5120
STEPS_TC = SPLIT // ROWS

SC_CORES = 2
SC_SUBCORES = 16
LANES = 16
NW = SC_CORES * SC_SUBCORES
COLS = N // NW
GRPS = COLS // LANES
CHUNK = 128
NCHUNK = (N - SPLIT) // CHUNK
SLOTS = B + 1


def _sc_segsum(ed_hbm, adj_hbm, out_hbm, edv, bufv, accv, sem0, sem1):
    cid = lax.axis_index("c")
    sid = lax.axis_index("s")
    w = sid * SC_CORES + cid

    pltpu.sync_copy(ed_hbm, edv)
    eds = [edv[pl.ds(k * LANES, LANES)][0] for k in range(B)]

    def seg_scalar(r):
        s = jnp.int32(0)
        for k in range(B):
            s = s + jnp.where(eds[k] <= r, jnp.int32(1), jnp.int32(0))
        return s

    z16 = jnp.zeros((LANES,), jnp.float32)
    for s in range(SLOTS):
        for g in range(GRPS):
            accv[s, pl.ds(g * LANES, LANES)] = z16

    col0 = w * COLS

    def src_slice(c):
        return adj_hbm.at[pl.ds(SPLIT + c * CHUNK, CHUNK), pl.ds(col0, COLS)]

    def start(bslot, c, sem):
        return pltpu.async_copy(src_slice(c), bufv.at[bslot], sem)

    def wait(bslot, c, sem):
        pltpu.make_async_copy(src_slice(c), bufv.at[bslot], sem).wait()

    sems = (sem0, sem1)

    start(0, 0, sems[0])

    def process(c, bslot):
        @pl.when(c + 1 < NCHUNK)
        def _pref():
            start(1 - bslot, c + 1, sems[1 - bslot])

        wait(bslot, c, sems[bslot])

        r0 = SPLIT + c * CHUNK
        s0 = seg_scalar(r0)
        s1 = seg_scalar(r0 + (CHUNK - 1))

        @pl.when(s0 == s1)
        def _fast():
            def rowf(jj, acc):
                j = jj * 4
                for u in range(4):
                    acc = tuple(
                        acc[g] + bufv[bslot, j + u, pl.ds(g * LANES, LANES)]
                        for g in range(GRPS)
                    )
                return acc

            acc = lax.fori_loop(
                0, CHUNK // 4, rowf,
                tuple(jnp.zeros((LANES,), jnp.float32) for _ in range(GRPS)),
            )
            for g in range(GRPS):
                accv[s0, pl.ds(g * LANES, LANES)] = (
                    accv[s0, pl.ds(g * LANES, LANES)] + acc[g]
                )

        @pl.when(s0 != s1)
        def _slow():
            def rows_(j, carry):
                s = seg_scalar(r0 + j)
                for g in range(GRPS):
                    accv[s, pl.ds(g * LANES, LANES)] = (
                        accv[s, pl.ds(g * LANES, LANES)]
                        + bufv[bslot, j, pl.ds(g * LANES, LANES)]
                    )
                return carry

            lax.fori_loop(0, CHUNK, rows_, 0)

    def outer(cc, carry):
        process(cc * 2, 0)
        process(cc * 2 + 1, 1)
        return carry

    lax.fori_loop(0, NCHUNK // 2, outer, 0)

    pltpu.sync_copy(accv, out_hbm.at[w])


def _tc_partial(ed_ref, x_ref, W_ref, b_ref, adj_ref, out_ref, hid_ref, acc_ref):
    i = pl.program_id(0)

    @pl.when(i == 0)
    def _init():
        acc_ref[...] = jnp.zeros_like(acc_ref)
        hid_ref[...] = (
            jnp.dot(x_ref[...], W_ref[...], preferred_element_type=jnp.float32)
            + b_ref[...]
        )

    base = i * ROWS
    s0 = jnp.int32(0)
    s1 = jnp.int32(0)
    for k in range(B):
        e = ed_ref[k]
        s0 = s0 + jnp.where(e <= base, 1, 0).astype(jnp.int32)
        s1 = s1 + jnp.where(e <= base + ROWS - 1, 1, 0).astype(jnp.int32)

    @pl.when(s0 == s1)
    def _fast():
        col = jnp.sum(adj_ref[...], axis=0, keepdims=True)
        acc_ref[pl.ds(s0, 1), :] += col

    @pl.when(s0 != s1)
    def _slow():
        rid = base + jax.lax.broadcasted_iota(jnp.int32, (ROWS, 1), 0)
        cnt = jnp.zeros((ROWS, 1), jnp.int32)
        for k in range(B):
            cnt = cnt + jnp.where(ed_ref[k] <= rid, 1, 0).astype(jnp.int32)
        tile = adj_ref[...]
        for s in range(B + 1):
            mask = (cnt == s).astype(jnp.float32)
            acc_ref[pl.ds(s, 1), :] += jnp.sum(tile * mask, axis=0, keepdims=True)

    @pl.when(i == STEPS_TC - 1)
    def _final():
        out_ref[...] = jnp.dot(
            acc_ref[0:B, :], hid_ref[...], preferred_element_type=jnp.float32
        )


def _tc_finish(parts_ref, tc_ref, hid_ref, out_ref):
    acc = tc_ref[...]
    for w in range(NW):
        acc = acc + jnp.dot(
            parts_ref[w, 0:B, :],
            hid_ref[w * COLS:(w + 1) * COLS, :],
            preferred_element_type=jnp.float32,
        )
    out_ref[...] = acc


def kernel(x, ed_idx, adj, W, b):
    ed_b = jnp.broadcast_to(ed_idx[:, None], (B, LANES)).reshape(B * LANES)
    ed16 = jnp.concatenate([ed_idx, jnp.full((8,), N, jnp.int32)])
    b2 = b.reshape(1, NC)

    sc = functools.partial(
        pl.kernel,
        out_type=jax.ShapeDtypeStruct((NW, SLOTS, COLS), jnp.float32),
        mesh=plsc.VectorSubcoreMesh(
            core_axis_name="c", subcore_axis_name="s",
            num_cores=SC_CORES, num_subcores=SC_SUBCORES,
        ),
        scratch_types=[
            pltpu.VMEM((B * LANES,), jnp.int32),
            pltpu.VMEM((2, CHUNK, COLS), jnp.float32),
            pltpu.VMEM((SLOTS, COLS), jnp.float32),
            pltpu.SemaphoreType.DMA,
            pltpu.SemaphoreType.DMA,
        ],
    )(_sc_segsum)
    parts = sc(ed_b, adj)

    grid_spec = pltpu.PrefetchScalarGridSpec(
        num_scalar_prefetch=1,
        grid=(STEPS_TC,),
        in_specs=[
            pl.BlockSpec((N, DIM), lambda i, *_: (0, 0)),
            pl.BlockSpec((DIM, NC), lambda i, *_: (0, 0)),
            pl.BlockSpec((1, NC), lambda i, *_: (0, 0)),
            pl.BlockSpec((ROWS, N), lambda i, *_: (i, 0)),
        ],
        out_specs=[
            pl.BlockSpec((B, NC), lambda i, *_: (0, 0)),
            pl.BlockSpec((N, NC), lambda i, *_: (0, 0)),
        ],
        scratch_shapes=[
            pltpu.VMEM((16, N), jnp.float32),
        ],
    )
    tc_out, hidden = pl.pallas_call(
        _tc_partial,
        grid_spec=grid_spec,
        out_shape=[
            jax.ShapeDtypeStruct((B, NC), jnp.float32),
            jax.ShapeDtypeStruct((N, NC), jnp.float32),
        ],
    )(ed16, x, W, b2, adj)

    return pl.pallas_call(
        _tc_finish,
        out_shape=jax.ShapeDtypeStruct((B, NC), jnp.float32),
    )(parts, tc_out, hidden)
